# baseline (speedup 1.0000x reference)
import jax
import jax.numpy as jnp
from jax import lax
from jax.experimental import pallas as pl
from jax.experimental.pallas import tpu as pltpu
from jax.experimental.pallas import tpu_sc as plsc

NC = 2
NS = 16
NW = NC * NS
L = 16
B = 16384
D = 64
BPW = B // NW
GSZ = 128
NG = BPW // GSZ

_MESH = dict(
    mesh=plsc.VectorSubcoreMesh(core_axis_name="c", subcore_axis_name="s"),
    compiler_params=pltpu.CompilerParams(use_tc_tiling_on_sc=False),
)


def _gather_body(idx2_h, tab_h, bias_h, out_h, idx, rows, brow, sem):
    wid = lax.axis_index("s") * NC + lax.axis_index("c")

    pltpu.sync_copy(idx2_h.at[wid], idx)

    copies = []
    for k in range(D):
        for j in range(NG):
            copies.append(pltpu.async_copy(
                tab_h.at[k].at[idx.at[j]],
                rows.at[k, pl.ds(j * GSZ, GSZ)], sem))
    for j in range(NG):
        copies.append(pltpu.async_copy(
            bias_h.at[0].at[idx.at[j]], brow.at[pl.ds(j * GSZ, GSZ)], sem))
    for c in copies:
        c.wait()

    pltpu.sync_copy(rows, out_h.at[wid, pl.ds(0, D)])
    pltpu.sync_copy(brow, out_h.at[wid, D])


def _combine_body(gu_h, gi_h, out_h, gu, gi, outv, sem):
    wid = lax.axis_index("s") * NC + lax.axis_index("c")

    pltpu.sync_copy(gu_h.at[wid], gu)
    pltpu.sync_copy(gi_h.at[wid], gi)

    def chunk(c, carry):
        off = c * L
        acc = gu[D, pl.ds(off, L)] + gi[D, pl.ds(off, L)]
        for k in range(D):
            acc = acc + gu[k, pl.ds(off, L)] * gi[k, pl.ds(off, L)]
        outv[pl.ds(off, L)] = acc
        return carry

    lax.fori_loop(0, BPW // L, chunk, 0)
    pltpu.sync_copy(outv, out_h.at[pl.ds(wid * BPW, BPW)])


@jax.jit
def _run(user2, item2, uet, iet, ubt, ibt):
    gather_scratch = [
        pltpu.VMEM((NG, GSZ), jnp.int32),
        pltpu.VMEM((D, BPW), jnp.float32),
        pltpu.VMEM((BPW,), jnp.float32),
        pltpu.SemaphoreType.DMA,
    ]
    stage_ty = jax.ShapeDtypeStruct((NW, D + 1, BPW), jnp.float32)

    user_k = pl.kernel(
        _gather_body, **_MESH, out_type=stage_ty, scratch_types=gather_scratch)
    item_k = pl.kernel(
        _gather_body, **_MESH, out_type=stage_ty, scratch_types=gather_scratch)
    gu = user_k(user2, uet, ubt)
    gi = item_k(item2, iet, ibt)

    combine_k = pl.kernel(
        _combine_body, **_MESH,
        out_type=jax.ShapeDtypeStruct((B,), jnp.float32),
        scratch_types=[
            pltpu.VMEM((D + 1, BPW), jnp.float32),
            pltpu.VMEM((D + 1, BPW), jnp.float32),
            pltpu.VMEM((BPW,), jnp.float32),
            pltpu.SemaphoreType.DMA,
        ],
    )
    return combine_k(gu, gi)


def kernel(user, item, user_embedding, item_embedding, user_bias, item_bias):
    n = user_embedding.shape[0] - 1
    uet = user_embedding.T[:, :n]
    iet = item_embedding.T[:, :n]
    ubt = user_bias.T[:, :n]
    ibt = item_bias.T[:, :n]
    user2 = user.astype(jnp.int32).reshape(NW, NG, GSZ)
    item2 = item.astype(jnp.int32).reshape(NW, NG, GSZ)
    return _run(user2, item2, uet, iet, ubt, ibt)


# speedup vs baseline: 12.9384x; 12.9384x over previous
import jax
import jax.numpy as jnp
from jax import lax
from jax.experimental import pallas as pl
from jax.experimental.pallas import tpu as pltpu
from jax.experimental.pallas import tpu_sc as plsc

NC = 2
NS = 16
NW = NC * NS
L = 16
B = 16384
D = 64
BPW = B // NW
GSZ = 128
NG = BPW // GSZ
HB = BPW // 2


def _bias_body(user_h, item_h, ub_h, ib_h, out_h, idx_u, idx_i, bu, bi, outv, sem):
    wid = lax.axis_index("s") * NC + lax.axis_index("c")

    pltpu.sync_copy(user_h.at[wid], idx_u)
    pltpu.sync_copy(item_h.at[wid], idx_i)

    copies = []
    for j in range(NG):
        dst = pl.ds(j * GSZ, GSZ)
        copies.append(pltpu.async_copy(ub_h.at[idx_u.at[j]], bu.at[dst], sem))
        copies.append(pltpu.async_copy(ib_h.at[idx_i.at[j]], bi.at[dst], sem))
    for c in copies:
        c.wait()

    def chunk(c, carry):
        off = c * L
        outv[pl.ds(off, L)] = bu[pl.ds(off, L)] + bi[pl.ds(off, L)]
        return carry

    lax.fori_loop(0, BPW // L, chunk, 0)
    pltpu.sync_copy(outv, out_h.at[pl.ds(wid * BPW, BPW)])


_GDN = lax.GatherDimensionNumbers(
    offset_dims=(), collapsed_slice_dims=(0,), start_index_map=(0,))


def _lane_perm(x, perm):
    return lax.gather(
        x, perm[:, None], dimension_numbers=_GDN, slice_sizes=(1,),
        mode=lax.GatherScatterMode.PROMISE_IN_BOUNDS)


def _main_body(user_h, item_h, ue_h, ie_h, bsum_h, out_h,
               idx_uv, idx_iv, shu, shi, idx_us, idx_is, gu, gi, bs, outv, sem):
    sid = lax.axis_index("s")
    wid = sid * NC + lax.axis_index("c")
    base = wid * BPW

    pltpu.sync_copy(user_h.at[pl.ds(base, BPW)], idx_uv)
    pltpu.sync_copy(item_h.at[pl.ds(base, BPW)], idx_iv)
    pltpu.sync_copy(idx_uv, shu.at[sid])
    pltpu.sync_copy(idx_iv, shi.at[sid])
    pltpu.sync_copy(shu.at[sid], idx_us)
    pltpu.sync_copy(shi.at[sid], idx_is)
    pltpu.sync_copy(bsum_h.at[pl.ds(base, BPW)], bs)

    lane = lax.iota(jnp.int32, 16)
    perms = [lane ^ (1 << p) for p in range(4)]
    FK = 16

    def half(h, _):
        row0 = h * HB

        def fire(c, carry):
            i0 = row0 + c * FK
            copies = []
            for b in range(FK):
                i = i0 + b
                u = idx_us[i]
                v = idx_is[i]
                copies.append(pltpu.async_copy(
                    ue_h.at[u], gu.at[i - row0], sem))
                copies.append(pltpu.async_copy(
                    ie_h.at[v], gi.at[i - row0], sem))
            for cp in copies:
                cp.wait()
            return carry

        lax.fori_loop(0, HB // FK, fire, 0)

        def chunk(c, carry):
            off = c * L
            res = bs[pl.ds(row0 + off, L)]
            for r in range(L):
                row = off + r
                acc = gu[row, pl.ds(0, L)] * gi[row, pl.ds(0, L)]
                for k in range(1, D // L):
                    acc = acc + gu[row, pl.ds(k * L, L)] * gi[row, pl.ds(k * L, L)]
                for p in perms:
                    acc = acc + _lane_perm(acc, p)
                res = jnp.where(lane == r, res + acc, res)
            outv[pl.ds(row0 + off, L)] = res
            return carry

        lax.fori_loop(0, HB // L, chunk, 0)
        return _

    for h in range(2):
        half(h, 0)

    pltpu.sync_copy(outv, out_h.at[pl.ds(base, BPW)])


@jax.jit
def _run(user2, item2, user, item, ue, ie, ub, ib):
    bias_k = pl.kernel(
        _bias_body,
        mesh=plsc.VectorSubcoreMesh(core_axis_name="c", subcore_axis_name="s"),
        compiler_params=pltpu.CompilerParams(use_tc_tiling_on_sc=False),
        out_type=jax.ShapeDtypeStruct((B,), jnp.float32),
        scratch_types=[
            pltpu.VMEM((NG, GSZ), jnp.int32),
            pltpu.VMEM((NG, GSZ), jnp.int32),
            pltpu.VMEM((BPW,), jnp.float32),
            pltpu.VMEM((BPW,), jnp.float32),
            pltpu.VMEM((BPW,), jnp.float32),
            pltpu.SemaphoreType.DMA,
        ],
    )
    bsum = bias_k(user2, item2, ub, ib)

    main_k = pl.kernel(
        _main_body,
        mesh=plsc.VectorSubcoreMesh(core_axis_name="c", subcore_axis_name="s"),
        compiler_params=pltpu.CompilerParams(use_tc_tiling_on_sc=True),
        out_type=jax.ShapeDtypeStruct((B,), jnp.float32),
        scratch_types=[
            pltpu.VMEM((BPW,), jnp.int32),
            pltpu.VMEM((BPW,), jnp.int32),
            pltpu.VMEM_SHARED((NS, BPW), jnp.int32),
            pltpu.VMEM_SHARED((NS, BPW), jnp.int32),
            pltpu.SMEM((BPW,), jnp.int32),
            pltpu.SMEM((BPW,), jnp.int32),
            pltpu.VMEM((HB, D), jnp.float32),
            pltpu.VMEM((HB, D), jnp.float32),
            pltpu.VMEM((BPW,), jnp.float32),
            pltpu.VMEM((BPW,), jnp.float32),
            pltpu.SemaphoreType.DMA,
        ],
    )
    return main_k(user, item, ue, ie, bsum)


def kernel(user, item, user_embedding, item_embedding, user_bias, item_bias):
    ub = user_bias.reshape(-1)
    ib = item_bias.reshape(-1)
    u32 = user.astype(jnp.int32)
    i32 = item.astype(jnp.int32)
    user2 = u32.reshape(NW, NG, GSZ)
    item2 = i32.reshape(NW, NG, GSZ)
    return _run(user2, item2, u32, i32, user_embedding, item_embedding, ub, ib)
